# baseline (speedup 1.0000x reference)
import functools
import jax
import jax.numpy as jnp
from jax import lax
from jax.experimental import pallas as pl
from jax.experimental.pallas import tpu as pltpu
from jax.experimental.pallas import tpu_sc as plsc

_NW = 32
_CP = 128
_NBUF = 4
_GRP = _NBUF * _CP


def _sc_kernel(total):
    pairs = total // 2
    p_per_w = pairs // _NW
    n_grp = p_per_w // _GRP
    mesh = plsc.VectorSubcoreMesh(core_axis_name="c", subcore_axis_name="s")

    @functools.partial(
        pl.kernel,
        out_type=jax.ShapeDtypeStruct((pairs, 128), jnp.float32),
        mesh=mesh,
        compiler_params=pltpu.CompilerParams(needs_layout_passes=False),
        scratch_types=(
            [pltpu.VMEM((_GRP,), jnp.int32)] * 2
            + [pltpu.VMEM((_GRP,), jnp.int32)]
            + [pltpu.VMEM((_CP, 128), jnp.float32)] * _NBUF
            + [pltpu.SemaphoreType.DMA] * (2 * _NBUF)
        ),
    )
    def k(basis_hbm, outcome_hbm, t2_hbm, out_hbm,
          b_stage, o_stage, idx_v, r0, r1, r2, r3,
          g0, g1, g2, g3, s0, s1, s2, s3):
        rows = (r0, r1, r2, r3)
        gsem = (g0, g1, g2, g3)
        ssem = (s0, s1, s2, s3)
        wid = lax.axis_index("s") * 2 + lax.axis_index("c")
        base = wid * p_per_w

        def group(g, _):
            off = base + g * _GRP
            pltpu.sync_copy(basis_hbm.at[pl.ds(off, _GRP)], b_stage)
            pltpu.sync_copy(outcome_hbm.at[pl.ds(off, _GRP)], o_stage)

            def ids16(kk, carry):
                s = pl.ds(kk * 16, 16)
                bw = b_stage[s]
                ow = o_stage[s]
                id_e = (bw & 0xFFFF) * 2 + (ow & 0xFFFF)
                id_o = (bw >> 16) * 2 + (ow >> 16)
                idx_v[s] = id_e * 6 + id_o
                return carry
            lax.fori_loop(0, _GRP // 16, ids16, 0)

            gathers = [
                pltpu.async_copy(
                    t2_hbm.at[idx_v.at[pl.ds(kk * _CP, _CP)]],
                    rows[kk], gsem[kk])
                for kk in range(_NBUF)
            ]
            scatters = []
            for kk in range(_NBUF):
                gathers[kk].wait()
                scatters.append(pltpu.async_copy(
                    rows[kk], out_hbm.at[pl.ds(off + kk * _CP, _CP)],
                    ssem[kk]))
            for kk in range(_NBUF):
                scatters[kk].wait()
            return _

        lax.fori_loop(0, n_grp, group, 0)

    return k


def kernel(basis, outcome, table):
    n, c = basis.shape
    total = n * c
    t2 = jnp.concatenate(
        [jnp.repeat(table, 6, axis=0), jnp.tile(table, (6, 1))], axis=1)
    bw = lax.bitcast_convert_type(
        basis.reshape(total // 2, 2).astype(jnp.int16), jnp.int32)
    ow = lax.bitcast_convert_type(
        outcome.reshape(total // 2, 2).astype(jnp.int16), jnp.int32)
    out = _sc_kernel(total)(bw, ow, t2)
    return out.reshape(n, c, 64)


# speedup vs baseline: 2.1980x; 2.1980x over previous
import functools
import jax
import jax.numpy as jnp
from jax import lax
from jax.experimental import pallas as pl
from jax.experimental.pallas import tpu as pltpu
from jax.experimental.pallas import tpu_sc as plsc

_N_SC = 4096
_NW = 32
_CP = 128
_NBUF = 4
_GRP = _NBUF * _CP
_R = 64


def _sc_kernel(total):
    pairs = total // 2
    p_per_w = pairs // _NW
    n_grp = p_per_w // _GRP
    mesh = plsc.VectorSubcoreMesh(core_axis_name="c", subcore_axis_name="s")

    @functools.partial(
        pl.kernel,
        out_type=jax.ShapeDtypeStruct((pairs, 128), jnp.float32),
        mesh=mesh,
        compiler_params=pltpu.CompilerParams(needs_layout_passes=False),
        scratch_types=(
            [pltpu.VMEM((_GRP,), jnp.int32)] * 2
            + [pltpu.VMEM((_GRP,), jnp.int32)]
            + [pltpu.VMEM((_CP, 128), jnp.float32)] * _NBUF
            + [pltpu.SemaphoreType.DMA] * (2 * _NBUF)
        ),
    )
    def k(basis_hbm, outcome_hbm, t2_hbm, out_hbm,
          b_stage, o_stage, idx_v, r0, r1, r2, r3,
          g0, g1, g2, g3, s0, s1, s2, s3):
        rows = (r0, r1, r2, r3)
        gsem = (g0, g1, g2, g3)
        ssem = (s0, s1, s2, s3)
        wid = lax.axis_index("s") * 2 + lax.axis_index("c")
        base = wid * p_per_w

        def group(g, _):
            off = base + g * _GRP
            pltpu.sync_copy(basis_hbm.at[pl.ds(off, _GRP)], b_stage)
            pltpu.sync_copy(outcome_hbm.at[pl.ds(off, _GRP)], o_stage)

            def ids16(kk, carry):
                s = pl.ds(kk * 16, 16)
                bw = b_stage[s]
                ow = o_stage[s]
                id_e = (bw & 0xFFFF) * 2 + (ow & 0xFFFF)
                id_o = (bw >> 16) * 2 + (ow >> 16)
                idx_v[s] = id_e * 6 + id_o
                return carry
            lax.fori_loop(0, _GRP // 16, ids16, 0)

            gathers = [
                pltpu.async_copy(
                    t2_hbm.at[idx_v.at[pl.ds(kk * _CP, _CP)]],
                    rows[kk], gsem[kk])
                for kk in range(_NBUF)
            ]
            scatters = []
            for kk in range(_NBUF):
                gathers[kk].wait()
                scatters.append(pltpu.async_copy(
                    rows[kk], out_hbm.at[pl.ds(off + kk * _CP, _CP)],
                    ssem[kk]))
            for kk in range(_NBUF):
                scatters[kk].wait()
            return _

        lax.fori_loop(0, n_grp, group, 0)

    return k


def _sc_part(basis, outcome, table):
    n, c = basis.shape
    total = n * c
    t2 = jnp.concatenate(
        [jnp.repeat(table, 6, axis=0), jnp.tile(table, (6, 1))], axis=1)
    bw = lax.bitcast_convert_type(
        basis.reshape(total // 2, 2).astype(jnp.int16), jnp.int32)
    ow = lax.bitcast_convert_type(
        outcome.reshape(total // 2, 2).astype(jnp.int16), jnp.int32)
    out = _sc_kernel(total)(bw, ow, t2)
    return out.reshape(n, c, 64)


def _tc_body(basis_ref, outcome_ref, table_ref, out_ref):
    r, c = basis_ref.shape
    ids = basis_ref[...] * 2 + outcome_ref[...]
    tab = table_ref[...]
    tok = lax.broadcasted_iota(jnp.int32, (6, c), 0)
    for g in range(r):
        row = jnp.broadcast_to(ids[g:g + 1, :], (6, c))
        onehot = (row == tok).astype(jnp.float32)
        res = lax.dot_general(onehot, tab, (((0,), (0,)), ((), ())),
                              preferred_element_type=jnp.float32)
        out_ref[pl.ds(g * c, c), :] = res


def _tc_part(basis, outcome, table):
    n, c = basis.shape
    total = n * c
    grid = (n // _R,)
    out = pl.pallas_call(
        _tc_body,
        grid=grid,
        in_specs=[
            pl.BlockSpec((_R, c), lambda i: (i, 0)),
            pl.BlockSpec((_R, c), lambda i: (i, 0)),
            pl.BlockSpec((6, 64), lambda i: (0, 0)),
        ],
        out_specs=pl.BlockSpec((_R * c, 64), lambda i: (i, 0)),
        out_shape=jax.ShapeDtypeStruct((total, 64), jnp.float32),
    )(basis, outcome, table)
    return out.reshape(n, c, 64)


def kernel(basis, outcome, table):
    out_sc = _sc_part(basis[:_N_SC], outcome[:_N_SC], table)
    out_tc = _tc_part(basis[_N_SC:], outcome[_N_SC:], table)
    return jnp.concatenate([out_sc, out_tc], axis=0)


# speedup vs baseline: 2.2060x; 1.0036x over previous
import functools
import jax
import jax.numpy as jnp
from jax import lax
from jax.experimental import pallas as pl
from jax.experimental.pallas import tpu as pltpu
from jax.experimental.pallas import tpu_sc as plsc

_N_SC = 4096
_NW = 32
_CP = 256
_NBUF = 2
_GRP = _NBUF * _CP
_R = 64


def _sc_kernel(total):
    pairs = total // 2
    p_per_w = pairs // _NW
    n_grp = p_per_w // _GRP
    mesh = plsc.VectorSubcoreMesh(core_axis_name="c", subcore_axis_name="s")

    @functools.partial(
        pl.kernel,
        out_type=jax.ShapeDtypeStruct((pairs, 128), jnp.float32),
        mesh=mesh,
        compiler_params=pltpu.CompilerParams(needs_layout_passes=False),
        scratch_types=(
            [pltpu.VMEM((_GRP,), jnp.int32)] * 2
            + [pltpu.VMEM((_GRP,), jnp.int32)]
            + [pltpu.VMEM((_CP, 128), jnp.float32)] * _NBUF
            + [pltpu.SemaphoreType.DMA] * (2 * _NBUF)
        ),
    )
    def k(basis_hbm, outcome_hbm, t2_hbm, out_hbm,
          b_stage, o_stage, idx_v, r0, r1,
          g0, g1, s0, s1):
        rows = (r0, r1)
        gsem = (g0, g1)
        ssem = (s0, s1)
        wid = lax.axis_index("s") * 2 + lax.axis_index("c")
        base = wid * p_per_w

        def group(g, _):
            off = base + g * _GRP
            pltpu.sync_copy(basis_hbm.at[pl.ds(off, _GRP)], b_stage)
            pltpu.sync_copy(outcome_hbm.at[pl.ds(off, _GRP)], o_stage)

            def ids16(kk, carry):
                s = pl.ds(kk * 16, 16)
                bw = b_stage[s]
                ow = o_stage[s]
                id_e = (bw & 0xFFFF) * 2 + (ow & 0xFFFF)
                id_o = (bw >> 16) * 2 + (ow >> 16)
                idx_v[s] = id_e * 6 + id_o
                return carry
            lax.fori_loop(0, _GRP // 16, ids16, 0)

            gathers = [
                pltpu.async_copy(
                    t2_hbm.at[idx_v.at[pl.ds(kk * _CP, _CP)]],
                    rows[kk], gsem[kk])
                for kk in range(_NBUF)
            ]
            scatters = []
            for kk in range(_NBUF):
                gathers[kk].wait()
                scatters.append(pltpu.async_copy(
                    rows[kk], out_hbm.at[pl.ds(off + kk * _CP, _CP)],
                    ssem[kk]))
            for kk in range(_NBUF):
                scatters[kk].wait()
            return _

        lax.fori_loop(0, n_grp, group, 0)

    return k


def _sc_part(basis, outcome, table):
    n, c = basis.shape
    total = n * c
    t2 = jnp.concatenate(
        [jnp.repeat(table, 6, axis=0), jnp.tile(table, (6, 1))], axis=1)
    bw = lax.bitcast_convert_type(
        basis.reshape(total // 2, 2).astype(jnp.int16), jnp.int32)
    ow = lax.bitcast_convert_type(
        outcome.reshape(total // 2, 2).astype(jnp.int16), jnp.int32)
    out = _sc_kernel(total)(bw, ow, t2)
    return out.reshape(n, c, 64)


def _tc_body(basis_ref, outcome_ref, table_ref, out_ref):
    r, c = basis_ref.shape
    ids = basis_ref[...] * 2 + outcome_ref[...]
    tab = table_ref[...]
    tok = lax.broadcasted_iota(jnp.int32, (6, c), 0)
    for g in range(r):
        row = jnp.broadcast_to(ids[g:g + 1, :], (6, c))
        onehot = (row == tok).astype(jnp.float32)
        res = lax.dot_general(onehot, tab, (((0,), (0,)), ((), ())),
                              preferred_element_type=jnp.float32)
        out_ref[pl.ds(g * c, c), :] = res


def _tc_part(basis, outcome, table):
    n, c = basis.shape
    total = n * c
    grid = (n // _R,)
    out = pl.pallas_call(
        _tc_body,
        grid=grid,
        in_specs=[
            pl.BlockSpec((_R, c), lambda i: (i, 0)),
            pl.BlockSpec((_R, c), lambda i: (i, 0)),
            pl.BlockSpec((6, 64), lambda i: (0, 0)),
        ],
        out_specs=pl.BlockSpec((_R * c, 64), lambda i: (i, 0)),
        out_shape=jax.ShapeDtypeStruct((total, 64), jnp.float32),
    )(basis, outcome, table)
    return out.reshape(n, c, 64)


def kernel(basis, outcome, table):
    out_sc = _sc_part(basis[:_N_SC], outcome[:_N_SC], table)
    out_tc = _tc_part(basis[_N_SC:], outcome[_N_SC:], table)
    return jnp.concatenate([out_sc, out_tc], axis=0)


# speedup vs baseline: 2.6146x; 1.1853x over previous
import functools
import jax
import jax.numpy as jnp
from jax import lax
from jax.experimental import pallas as pl
from jax.experimental.pallas import tpu as pltpu
from jax.experimental.pallas import tpu_sc as plsc

_N_SC = 2048
_NW = 32
_CP = 128
_NBUF = 2
_GRP = _NBUF * _CP
_R = 64


def _sc_kernel(total):
    pairs = total // 2
    p_per_w = pairs // _NW
    n_grp = p_per_w // _GRP
    mesh = plsc.VectorSubcoreMesh(core_axis_name="c", subcore_axis_name="s")

    @functools.partial(
        pl.kernel,
        out_type=jax.ShapeDtypeStruct((pairs, 128), jnp.float32),
        mesh=mesh,
        compiler_params=pltpu.CompilerParams(needs_layout_passes=False),
        scratch_types=(
            [pltpu.VMEM((_GRP,), jnp.int32)] * 2
            + [pltpu.VMEM((_GRP,), jnp.int32)]
            + [pltpu.VMEM((_CP, 128), jnp.float32)] * _NBUF
            + [pltpu.SemaphoreType.DMA] * (2 * _NBUF)
        ),
    )
    def k(basis_hbm, outcome_hbm, t2_hbm, out_hbm,
          b_stage, o_stage, idx_v, r0, r1,
          g0, g1, s0, s1):
        rows = (r0, r1)
        gsem = (g0, g1)
        ssem = (s0, s1)
        wid = lax.axis_index("s") * 2 + lax.axis_index("c")
        base = wid * p_per_w

        def group(g, _):
            off = base + g * _GRP
            pltpu.sync_copy(basis_hbm.at[pl.ds(off, _GRP)], b_stage)
            pltpu.sync_copy(outcome_hbm.at[pl.ds(off, _GRP)], o_stage)

            def ids16(kk, carry):
                s = pl.ds(kk * 16, 16)
                bw = b_stage[s]
                ow = o_stage[s]
                id_e = (bw & 0xFFFF) * 2 + (ow & 0xFFFF)
                id_o = (bw >> 16) * 2 + (ow >> 16)
                idx_v[s] = id_e * 6 + id_o
                return carry
            lax.fori_loop(0, _GRP // 16, ids16, 0)

            gathers = [
                pltpu.async_copy(
                    t2_hbm.at[idx_v.at[pl.ds(kk * _CP, _CP)]],
                    rows[kk], gsem[kk])
                for kk in range(_NBUF)
            ]
            scatters = []
            for kk in range(_NBUF):
                gathers[kk].wait()
                scatters.append(pltpu.async_copy(
                    rows[kk], out_hbm.at[pl.ds(off + kk * _CP, _CP)],
                    ssem[kk]))
            for kk in range(_NBUF):
                scatters[kk].wait()
            return _

        lax.fori_loop(0, n_grp, group, 0)

    return k


def _sc_part(basis, outcome, table):
    n, c = basis.shape
    total = n * c
    t2 = jnp.concatenate(
        [jnp.repeat(table, 6, axis=0), jnp.tile(table, (6, 1))], axis=1)
    bw = lax.bitcast_convert_type(
        basis.reshape(total // 2, 2).astype(jnp.int16), jnp.int32)
    ow = lax.bitcast_convert_type(
        outcome.reshape(total // 2, 2).astype(jnp.int16), jnp.int32)
    out = _sc_kernel(total)(bw, ow, t2)
    return out.reshape(n, c, 64)


def _tc_body(basis_ref, outcome_ref, table_ref, out_ref):
    r, c = basis_ref.shape
    ids = basis_ref[...] * 2 + outcome_ref[...]
    tab = table_ref[...]
    tok = lax.broadcasted_iota(jnp.int32, (6, c), 0)
    for g in range(r):
        row = jnp.broadcast_to(ids[g:g + 1, :], (6, c))
        onehot = (row == tok).astype(jnp.float32)
        res = lax.dot_general(onehot, tab, (((0,), (0,)), ((), ())),
                              preferred_element_type=jnp.float32)
        out_ref[pl.ds(g * c, c), :] = res


def _tc_part(basis, outcome, table):
    n, c = basis.shape
    total = n * c
    grid = (n // _R,)
    out = pl.pallas_call(
        _tc_body,
        grid=grid,
        in_specs=[
            pl.BlockSpec((_R, c), lambda i: (i, 0)),
            pl.BlockSpec((_R, c), lambda i: (i, 0)),
            pl.BlockSpec((6, 64), lambda i: (0, 0)),
        ],
        out_specs=pl.BlockSpec((_R * c, 64), lambda i: (i, 0)),
        out_shape=jax.ShapeDtypeStruct((total, 64), jnp.float32),
    )(basis, outcome, table)
    return out.reshape(n, c, 64)


def kernel(basis, outcome, table):
    out_sc = _sc_part(basis[:_N_SC], outcome[:_N_SC], table)
    out_tc = _tc_part(basis[_N_SC:], outcome[_N_SC:], table)
    return jnp.concatenate([out_sc, out_tc], axis=0)
